# baseline (speedup 1.0000x reference)
import functools

import jax
import jax.numpy as jnp
from jax import lax
from jax.experimental import pallas as pl
from jax.experimental.pallas import tpu as pltpu
from jax.experimental.pallas import tpu_sc as plsc

_BETA = 25.0
_NC = 2
_NS = 16
_NW = _NC * _NS
_LANES = 16
_SC_ROWS = 4096
_TC_BLK = 512


def _make_sc_body(D, K, rows):
    nchunk = K // _LANES

    def body(x_hbm, pairs_hbm, out_hbm, x_v, pairs_v, out_v):
        wid = lax.axis_index("s") * _NC + lax.axis_index("c")
        base = wid * rows

        pltpu.sync_copy(pairs_hbm, pairs_v)
        pltpu.sync_copy(x_hbm.at[pl.ds(base * D, rows * D)], x_v)

        lane = lax.iota(jnp.int32, _LANES)
        idx_i = []
        idx_j = []
        for c in range(nchunk):
            kvec = (c * _LANES + lane) * 2
            idx_i.append(plsc.load_gather(pairs_v, [kvec]))
            idx_j.append(plsc.load_gather(pairs_v, [kvec + 1]))

        @plsc.parallel_loop(0, rows, 1, unroll=8)
        def _row(r):
            xrow = x_v.at[pl.ds(r * D, D)]
            orow = out_v.at[pl.ds(r * K, K)]
            for c in range(nchunk):
                xi = plsc.load_gather(xrow, [idx_i[c]])
                xj = plsc.load_gather(xrow, [idx_j[c]])
                z = (xj - xi) * _BETA
                orow[pl.ds(c * _LANES, _LANES)] = 1.0 / (1.0 + jnp.exp(z))

        pltpu.sync_copy(out_v, out_hbm.at[pl.ds(base * K, rows * K)])

    return body


def _tc_body(pairs_ref, x_ref, o_ref):
    D = x_ref.shape[1]
    K = pairs_ref.shape[0]
    p = pairs_ref[...]
    d_iota = lax.broadcasted_iota(jnp.int32, (D, K), 0)
    p0 = jnp.broadcast_to(p[:, 0][None, :], (D, K))
    p1 = jnp.broadcast_to(p[:, 1][None, :], (D, K))
    sel = jnp.where(d_iota == p0, 1.0, 0.0) - jnp.where(d_iota == p1, 1.0, 0.0)
    diff = jnp.dot(x_ref[...], sel, preferred_element_type=jnp.float32,
                   precision=lax.Precision.HIGHEST)
    o_ref[...] = 1.0 / (1.0 + jnp.exp(-_BETA * diff))


def kernel(x, pairs):
    B, D = x.shape
    K = pairs.shape[0]
    sc_rows = _SC_ROWS
    tc_rows = B - sc_rows
    rows_per_tile = sc_rows // _NW

    sc_run = pl.kernel(
        _make_sc_body(D, K, rows_per_tile),
        out_type=jax.ShapeDtypeStruct((sc_rows * K,), jnp.float32),
        mesh=plsc.VectorSubcoreMesh(core_axis_name="c", subcore_axis_name="s"),
        compiler_params=pltpu.CompilerParams(needs_layout_passes=False),
        scratch_types=[
            pltpu.VMEM((rows_per_tile * D,), jnp.float32),
            pltpu.VMEM((K * 2,), jnp.int32),
            pltpu.VMEM((rows_per_tile * K,), jnp.float32),
        ],
    )

    blk0 = sc_rows // _TC_BLK
    tc_run = pl.pallas_call(
        _tc_body,
        grid=(tc_rows // _TC_BLK,),
        in_specs=[
            pl.BlockSpec((K, 2), lambda i: (0, 0)),
            pl.BlockSpec((_TC_BLK, D), lambda i: (i + blk0, 0)),
        ],
        out_specs=pl.BlockSpec((_TC_BLK, K), lambda i: (i, 0)),
        out_shape=jax.ShapeDtypeStruct((tc_rows, K), jnp.float32),
    )

    out_sc = sc_run(x.reshape(B * D), pairs.reshape(K * 2))
    out_tc = tc_run(pairs, x)
    return jnp.concatenate([out_sc.reshape(sc_rows, K), out_tc], axis=0)


# speedup vs baseline: 1.2131x; 1.2131x over previous
import functools

import jax
import jax.numpy as jnp
from jax import lax
from jax.experimental import pallas as pl
from jax.experimental.pallas import tpu as pltpu
from jax.experimental.pallas import tpu_sc as plsc

_BETA = 25.0
_NC = 2
_NS = 16
_NW = _NC * _NS
_LANES = 16
_NHALF = 2


def _make_body(B, D, K):
    rows = B // _NW
    half = rows // _NHALF
    nchunk = K // _LANES

    def body(x_hbm, pairs_hbm, out_hbm,
             x_v0, x_v1, o_v0, o_v1, pairs_v,
             si0, si1, so0, so1):
        xbufs = (x_v0, x_v1)
        obufs = (o_v0, o_v1)
        sin = (si0, si1)
        sout = (so0, so1)

        wid = lax.axis_index("s") * _NC + lax.axis_index("c")
        base = wid * rows

        in_h = []
        for h in range(_NHALF):
            src = x_hbm.at[pl.ds((base + h * half) * D, half * D)]
            in_h.append(pltpu.async_copy(src, xbufs[h], sin[h]))

        pltpu.sync_copy(pairs_hbm, pairs_v)
        lane = lax.iota(jnp.int32, _LANES)
        idx_i = []
        idx_j = []
        for c in range(nchunk):
            kvec = (c * _LANES + lane) * 2
            idx_i.append(plsc.load_gather(pairs_v, [kvec]))
            idx_j.append(plsc.load_gather(pairs_v, [kvec + 1]))

        out_h = []
        for h in range(_NHALF):
            in_h[h].wait()
            x_v = xbufs[h]
            o_v = obufs[h]

            @plsc.parallel_loop(0, half, 1, unroll=8)
            def _row(r):
                xrow = x_v.at[pl.ds(r * D, D)]
                orow = o_v.at[pl.ds(r * K, K)]
                for c in range(nchunk):
                    xi = plsc.load_gather(xrow, [idx_i[c]])
                    xj = plsc.load_gather(xrow, [idx_j[c]])
                    z = (xj - xi) * _BETA
                    orow[pl.ds(c * _LANES, _LANES)] = 1.0 / (1.0 + jnp.exp(z))

            dst = out_hbm.at[pl.ds((base + h * half) * K, half * K)]
            out_h.append(pltpu.async_copy(o_v, dst, sout[h]))

        for h in out_h:
            h.wait()

    return body


def kernel(x, pairs):
    B, D = x.shape
    K = pairs.shape[0]
    rows = B // _NW
    half = rows // _NHALF
    run = pl.kernel(
        _make_body(B, D, K),
        out_type=jax.ShapeDtypeStruct((B * K,), jnp.float32),
        mesh=plsc.VectorSubcoreMesh(core_axis_name="c", subcore_axis_name="s"),
        compiler_params=pltpu.CompilerParams(needs_layout_passes=False),
        scratch_types=[
            pltpu.VMEM((half * D,), jnp.float32),
            pltpu.VMEM((half * D,), jnp.float32),
            pltpu.VMEM((half * K,), jnp.float32),
            pltpu.VMEM((half * K,), jnp.float32),
            pltpu.VMEM((K * 2,), jnp.int32),
            pltpu.SemaphoreType.DMA,
            pltpu.SemaphoreType.DMA,
            pltpu.SemaphoreType.DMA,
            pltpu.SemaphoreType.DMA,
        ],
    )
    out = run(x.reshape(B * D), pairs.reshape(K * 2))
    return out.reshape(B, K)
